# baseline (speedup 1.0000x reference)
import functools

import jax
import jax.numpy as jnp
from jax import lax
from jax.experimental import pallas as pl
from jax.experimental.pallas import tpu as pltpu
from jax.experimental.pallas import tpu_sc as plsc

E = 320000
N = 10000
NPAD = 10240
D = 128
W = D // 2
L = 16
NC = 2
NS = 16
NW = NC * NS
PW = E // NW
CH = 80
NBUF = 4
CPAD = 128
IPADW = (CPAD + NBUF - 1) * CH
NGRP = CH // L
WCH = 16

_mesh = plsc.VectorSubcoreMesh(core_axis_name="c", subcore_axis_name="s")


@functools.partial(
    pl.kernel,
    mesh=_mesh,
    compiler_params=pltpu.CompilerParams(needs_layout_passes=False,
                                         use_tc_tiling_on_sc=False),
    out_type=jax.ShapeDtypeStruct((E,), jnp.float32),
    scratch_types=[
        pltpu.VMEM((IPADW,), jnp.int32),
        pltpu.VMEM((IPADW,), jnp.int32),
        *([pltpu.VMEM((CH, W), jnp.int32)] * (2 * NBUF)),
        pltpu.VMEM((IPADW,), jnp.float32),
        *([pltpu.SemaphoreType.DMA] * NBUF),
    ],
)
def _edge_dot(z_hbm, src_hbm, dst_hbm, out_hbm, sidx_v, didx_v,
              sr0, dr0, sr1, dr1, sr2, dr2, sr3, dr3, out_v,
              sem0, sem1, sem2, sem3):
    cid = lax.axis_index("c")
    sid = lax.axis_index("s")
    wid = sid * NC + cid
    base = wid * PW
    lanes = lax.iota(jnp.int32, L)
    srows = (sr0, sr1, sr2, sr3)
    drows = (dr0, dr1, dr2, dr3)
    sems = (sem0, sem1, sem2, sem3)

    zeros16 = jnp.zeros((L,), jnp.int32)
    for t in range((IPADW - PW) // L):
        sidx_v[pl.ds(PW + t * L, L)] = zeros16
        didx_v[pl.ds(PW + t * L, L)] = zeros16
    pltpu.sync_copy(src_hbm.at[pl.ds(base, PW)], sidx_v.at[pl.ds(0, PW)])
    pltpu.sync_copy(dst_hbm.at[pl.ds(base, PW)], didx_v.at[pl.ds(0, PW)])

    def fetch(c, b):
        sl = pl.ds(c * CH, CH)
        pltpu.async_copy(z_hbm.at[sidx_v.at[sl]], srows[b], sems[b])
        pltpu.async_copy(z_hbm.at[didx_v.at[sl]], drows[b], sems[b])

    def drain(b):
        pltpu.make_async_copy(z_hbm.at[sidx_v.at[pl.ds(0, CH)]],
                              srows[b], sems[b]).wait()
        pltpu.make_async_copy(z_hbm.at[didx_v.at[pl.ds(0, CH)]],
                              drows[b], sems[b]).wait()

    def compute(c, b):
        def grp_body(g, gcarry):
            rows16 = g * L + lanes

            def wchunk(k, carry):
                col, acc_a, acc_b = carry
                for w in range(0, WCH, 2):
                    ws0 = plsc.load_gather(srows[b], [rows16, col])
                    wd0 = plsc.load_gather(drows[b], [rows16, col])
                    col1 = lax.bitwise_and(col + 1, W - 1)
                    ws1 = plsc.load_gather(srows[b], [rows16, col1])
                    wd1 = plsc.load_gather(drows[b], [rows16, col1])
                    col = lax.bitwise_and(col + 2, W - 1)
                    pr = (plsc.bitcast(ws0, jnp.bfloat16)
                          * plsc.bitcast(wd0, jnp.bfloat16)
                          + plsc.bitcast(ws1, jnp.bfloat16)
                          * plsc.bitcast(wd1, jnp.bfloat16))
                    pa, pb = plsc.unpack(
                        pr, format=plsc.PackFormat.INTERLEAVED,
                        preferred_element_type=jnp.float32)
                    acc_a = acc_a + pa
                    acc_b = acc_b + pb
                return col, acc_a, acc_b

            zero = jnp.zeros((L,), jnp.float32)
            _, acc_a, acc_b = lax.fori_loop(0, W // WCH, wchunk,
                                            (lanes, zero, zero))
            out_v[pl.ds(c * CH + g * L, L)] = acc_a + acc_b
            return gcarry

        lax.fori_loop(0, NGRP, grp_body, 0)

    for b in range(NBUF - 1):
        fetch(b, b)

    def quad_body(p, carry):
        c0 = NBUF * p
        for b in range(NBUF):
            fetch(c0 + b + NBUF - 1, (b + NBUF - 1) % NBUF)
            drain(b)
            compute(c0 + b, b)
        return carry

    lax.fori_loop(0, CPAD // NBUF, quad_body, 0)

    pltpu.sync_copy(out_v.at[pl.ds(0, PW)], out_hbm.at[pl.ds(base, PW)])


def kernel(z, edge_label_index):
    idx = edge_label_index.astype(jnp.int32)
    zw = lax.bitcast_convert_type(
        z.astype(jnp.bfloat16).reshape(N, W, 2), jnp.int32)
    zw = jnp.pad(zw, ((0, NPAD - N), (0, 0)))
    return _edge_dot(zw, idx[0], idx[1])


# speedup vs baseline: 1.1969x; 1.1969x over previous
import functools

import jax
import jax.numpy as jnp
from jax import lax
from jax.experimental import pallas as pl
from jax.experimental.pallas import tpu as pltpu
from jax.experimental.pallas import tpu_sc as plsc

E = 320000
N = 10000
NPAD = 10240
D = 128
W = D // 2
L = 16
NC = 2
NS = 16
NW = NC * NS
PW = E // NW
CH = 128
CPAD = 80
IPADW = (CPAD + 1) * CH
NGRP = CH // L
WCH = 16

_mesh = plsc.VectorSubcoreMesh(core_axis_name="c", subcore_axis_name="s")


@functools.partial(
    pl.kernel,
    mesh=_mesh,
    compiler_params=pltpu.CompilerParams(needs_layout_passes=False,
                                         use_tc_tiling_on_sc=False),
    out_type=jax.ShapeDtypeStruct((E,), jnp.float32),
    scratch_types=[
        pltpu.VMEM((IPADW,), jnp.int32),
        pltpu.VMEM((IPADW,), jnp.int32),
        pltpu.VMEM((CH, W), jnp.int32),
        pltpu.VMEM((CH, W), jnp.int32),
        pltpu.VMEM((CH, W), jnp.int32),
        pltpu.VMEM((CH, W), jnp.int32),
        pltpu.VMEM((IPADW,), jnp.float32),
        pltpu.SemaphoreType.DMA,
        pltpu.SemaphoreType.DMA,
    ],
)
def _edge_dot(z_hbm, src_hbm, dst_hbm, out_hbm, sidx_v, didx_v,
              sr0, dr0, sr1, dr1, out_v, sem0, sem1):
    cid = lax.axis_index("c")
    sid = lax.axis_index("s")
    wid = sid * NC + cid
    base = wid * PW
    lanes = lax.iota(jnp.int32, L)
    srows = (sr0, sr1)
    drows = (dr0, dr1)
    sems = (sem0, sem1)

    zeros16 = jnp.zeros((L,), jnp.int32)
    for t in range((IPADW - PW) // L):
        sidx_v[pl.ds(PW + t * L, L)] = zeros16
        didx_v[pl.ds(PW + t * L, L)] = zeros16
    pltpu.sync_copy(src_hbm.at[pl.ds(base, PW)], sidx_v.at[pl.ds(0, PW)])
    pltpu.sync_copy(dst_hbm.at[pl.ds(base, PW)], didx_v.at[pl.ds(0, PW)])

    def fetch(c, b):
        sl = pl.ds(c * CH, CH)
        pltpu.async_copy(z_hbm.at[sidx_v.at[sl]], srows[b], sems[b])
        pltpu.async_copy(z_hbm.at[didx_v.at[sl]], drows[b], sems[b])

    def drain(b):
        pltpu.make_async_copy(z_hbm.at[sidx_v.at[pl.ds(0, CH)]],
                              srows[b], sems[b]).wait()
        pltpu.make_async_copy(z_hbm.at[didx_v.at[pl.ds(0, CH)]],
                              drows[b], sems[b]).wait()

    def compute(c, b):
        def grp_body(g, gcarry):
            rows16 = g * L + lanes

            def wchunk(k, carry):
                col, acc_a, acc_b = carry
                for w in range(0, WCH, 2):
                    ws0 = plsc.load_gather(srows[b], [rows16, col])
                    wd0 = plsc.load_gather(drows[b], [rows16, col])
                    col1 = lax.bitwise_and(col + 1, W - 1)
                    ws1 = plsc.load_gather(srows[b], [rows16, col1])
                    wd1 = plsc.load_gather(drows[b], [rows16, col1])
                    col = lax.bitwise_and(col + 2, W - 1)
                    pr = (plsc.bitcast(ws0, jnp.bfloat16)
                          * plsc.bitcast(wd0, jnp.bfloat16)
                          + plsc.bitcast(ws1, jnp.bfloat16)
                          * plsc.bitcast(wd1, jnp.bfloat16))
                    pa, pb = plsc.unpack(
                        pr, format=plsc.PackFormat.INTERLEAVED,
                        preferred_element_type=jnp.float32)
                    acc_a = acc_a + pa
                    acc_b = acc_b + pb
                return col, acc_a, acc_b

            zero = jnp.zeros((L,), jnp.float32)
            _, acc_a, acc_b = lax.fori_loop(0, W // WCH, wchunk,
                                            (lanes, zero, zero))
            out_v[pl.ds(c * CH + g * L, L)] = acc_a + acc_b
            return gcarry

        lax.fori_loop(0, NGRP, grp_body, 0)

    fetch(0, 0)

    def pair_body(p, carry):
        c0 = 2 * p
        fetch(c0 + 1, 1)
        drain(0)
        compute(c0, 0)
        fetch(c0 + 2, 0)
        drain(1)
        compute(c0 + 1, 1)
        return carry

    lax.fori_loop(0, CPAD // 2, pair_body, 0)

    pltpu.sync_copy(out_v.at[pl.ds(0, PW)], out_hbm.at[pl.ds(base, PW)])


def kernel(z, edge_label_index):
    idx = edge_label_index.astype(jnp.int32)
    zw = lax.bitcast_convert_type(
        z.astype(jnp.bfloat16).reshape(N, W, 2), jnp.int32)
    zw = jnp.pad(zw, ((0, NPAD - N), (0, 0)))
    return _edge_dot(zw, idx[0], idx[1])


# speedup vs baseline: 4.2467x; 3.5479x over previous
import functools

import jax
import jax.numpy as jnp
from jax import lax
from jax.experimental import pallas as pl
from jax.experimental.pallas import tpu as pltpu
from jax.experimental.pallas import tpu_sc as plsc

E = 320000
N = 10000
NPAD = 10240
D = 128
W = D // 2
L = 16
NC = 2
NS = 16
NW = NC * NS
PW = E // NW
CH = 80
NCHUNK = PW // CH
NGRP = CH // L

_mesh = plsc.VectorSubcoreMesh(core_axis_name="c", subcore_axis_name="s")


@functools.partial(
    pl.kernel,
    mesh=_mesh,
    compiler_params=pltpu.CompilerParams(needs_layout_passes=False,
                                         use_tc_tiling_on_sc=False),
    out_type=jax.ShapeDtypeStruct((E,), jnp.float32),
    scratch_types=[
        pltpu.VMEM((PW,), jnp.int32),
        pltpu.VMEM((PW,), jnp.int32),
        pltpu.VMEM((CH, W), jnp.int32),
        pltpu.VMEM((CH, W), jnp.int32),
        pltpu.VMEM((CH, W), jnp.int32),
        pltpu.VMEM((CH, W), jnp.int32),
        pltpu.VMEM((PW,), jnp.float32),
        pltpu.SemaphoreType.DMA,
        pltpu.SemaphoreType.DMA,
    ],
)
def _edge_dot(z_hbm, src_hbm, dst_hbm, out_hbm, sidx_v,
              didx_v, srows0_v, drows0_v, srows1_v, drows1_v, out_v,
              sem0, sem1):
    cid = lax.axis_index("c")
    sid = lax.axis_index("s")
    wid = sid * NC + cid
    base = wid * PW
    lanes = lax.iota(jnp.int32, L)
    srows = (srows0_v, srows1_v)
    drows = (drows0_v, drows1_v)
    sems = (sem0, sem1)

    pltpu.sync_copy(src_hbm.at[pl.ds(base, PW)], sidx_v)
    pltpu.sync_copy(dst_hbm.at[pl.ds(base, PW)], didx_v)

    def fetch(c, b):
        sl = pl.ds(c * CH, CH)
        pltpu.async_copy(z_hbm.at[sidx_v.at[sl]], srows[b], sems[b])
        pltpu.async_copy(z_hbm.at[didx_v.at[sl]], drows[b], sems[b])

    def drain(b):
        pltpu.make_async_copy(z_hbm.at[sidx_v.at[pl.ds(0, CH)]],
                              srows[b], sems[b]).wait()
        pltpu.make_async_copy(z_hbm.at[didx_v.at[pl.ds(0, CH)]],
                              drows[b], sems[b]).wait()

    WCH = 32

    def compute(c, b):
        def grp_body(g, gcarry):
            rows16 = g * L + lanes

            def wchunk(k, carry):
                col, acc_a, acc_b = carry
                for w in range(0, WCH, 2):
                    ws0 = plsc.load_gather(srows[b], [rows16, col])
                    wd0 = plsc.load_gather(drows[b], [rows16, col])
                    col1 = lax.bitwise_and(col + 1, W - 1)
                    ws1 = plsc.load_gather(srows[b], [rows16, col1])
                    wd1 = plsc.load_gather(drows[b], [rows16, col1])
                    col = lax.bitwise_and(col + 2, W - 1)
                    pr = (plsc.bitcast(ws0, jnp.bfloat16)
                          * plsc.bitcast(wd0, jnp.bfloat16)
                          + plsc.bitcast(ws1, jnp.bfloat16)
                          * plsc.bitcast(wd1, jnp.bfloat16))
                    pa, pb = plsc.unpack(
                        pr, format=plsc.PackFormat.INTERLEAVED,
                        preferred_element_type=jnp.float32)
                    acc_a = acc_a + pa
                    acc_b = acc_b + pb
                return col, acc_a, acc_b

            zero = jnp.zeros((L,), jnp.float32)
            _, acc_a, acc_b = lax.fori_loop(0, W // WCH, wchunk,
                                            (lanes, zero, zero))
            out_v[pl.ds(c * CH + g * L, L)] = acc_a + acc_b
            return gcarry

        lax.fori_loop(0, NGRP, grp_body, 0)

    fetch(0, 0)

    def pair_body(p, carry):
        c0 = 2 * p
        fetch(c0 + 1, 1)
        drain(0)
        compute(c0, 0)
        fetch(c0 + 2, 0)
        drain(1)
        compute(c0 + 1, 1)
        return carry

    lax.fori_loop(0, (NCHUNK - 1) // 2, pair_body, 0)

    drain(0)
    compute(NCHUNK - 1, 0)

    pltpu.sync_copy(out_v, out_hbm.at[pl.ds(base, PW)])


def kernel(z, edge_label_index):
    idx = edge_label_index.astype(jnp.int32)
    zw = lax.bitcast_convert_type(
        z.astype(jnp.bfloat16).reshape(N, W, 2), jnp.int32)
    zw = jnp.pad(zw, ((0, NPAD - N), (0, 0)))
    return _edge_dot(zw, idx[0], idx[1])
